# baseline (speedup 1.0000x reference)
import functools

import jax
import jax.numpy as jnp
from jax import lax
from jax.experimental import pallas as pl
from jax.experimental.pallas import tpu as pltpu
from jax.experimental.pallas import tpu_sc as plsc

_B = 16384
_L = 50
_EMB = 64
_NW = 32
_SPW = _B // _NW
_SEG = 256
_NSEG = _SPW // _SEG
_NBUF = 4
_LANES = 16
_G = _EMB // _LANES
_UNROLL = 10

_mesh = plsc.VectorSubcoreMesh(core_axis_name="c", subcore_axis_name="s")


@functools.partial(
    pl.kernel,
    mesh=_mesh,
    out_type=jax.ShapeDtypeStruct((_B, 2 * _EMB), jnp.float32),
    scratch_types=[
        pltpu.VMEM((_SEG, _L), jnp.int32),
        pltpu.VMEM((_NBUF, _L, _EMB), jnp.float32),
        pltpu.VMEM((_SPW, 2 * _EMB), jnp.float32),
        pltpu.SemaphoreType.DMA,
        pltpu.SemaphoreType.DMA,
        pltpu.SemaphoreType.DMA,
        pltpu.SemaphoreType.DMA,
    ],
    compiler_params=pltpu.CompilerParams(
        use_tc_tiling_on_sc=False, needs_layout_passes=False),
)
def _pool_kernel(idx_p, idx_h, tab_p, tab_h, out,
                 idx_v, rows, pooled, sem0, sem1, sem2, sem3):
    sems = (sem0, sem1, sem2, sem3)
    wid = lax.axis_index("s") * 2 + lax.axis_index("c")
    base = wid * _SPW

    def accumulate(rows_ref, dst_row, col):
        def rbody(r0, carry):
            accs = carry
            for k in range(_UNROLL):
                r = r0 * _UNROLL + k
                accs = tuple(
                    accs[g] + rows_ref[r, pl.ds(g * _LANES, _LANES)]
                    for g in range(_G)
                )
            return accs
        z = jnp.zeros((_LANES,), jnp.float32)
        acc = lax.fori_loop(0, _L // _UNROLL, rbody, (z,) * _G)
        for g in range(_G):
            pooled[dst_row, pl.ds(col + g * _LANES, _LANES)] = acc[g]

    for half, (idx_hbm, tab) in enumerate(((idx_p, tab_p), (idx_h, tab_h))):
        col = half * _EMB

        def seg_body(seg, _):
            s0 = base + seg * _SEG
            pltpu.sync_copy(idx_hbm.at[pl.ds(s0, _SEG)], idx_v)
            for b in range(_NBUF):
                pltpu.async_copy(tab.at[idx_v.at[b]], rows.at[b], sems[b])

            def j_body(j, _):
                for b in range(_NBUF):
                    smp = _NBUF * j + b
                    pltpu.make_async_copy(
                        tab.at[idx_v.at[smp]], rows.at[b], sems[b]).wait()
                    accumulate(rows.at[b], seg * _SEG + smp, col)

                    @pl.when(smp + _NBUF < _SEG)
                    def _():
                        pltpu.async_copy(
                            tab.at[idx_v.at[smp + _NBUF]], rows.at[b], sems[b])
                return 0

            lax.fori_loop(0, _SEG // _NBUF, j_body, 0)
            return 0

        lax.fori_loop(0, _NSEG, seg_body, 0)

    pltpu.sync_copy(pooled, out.at[pl.ds(base, _SPW)])


def _linear_body(x_ref, w_ref, b_ref, o_ref):
    o_ref[...] = (
        jnp.dot(x_ref[...], w_ref[...], preferred_element_type=jnp.float32)
        + b_ref[...]
    )


def kernel(premise_indices, hypothesis_indices, table_prem, table_hypo, W, b):
    pi = premise_indices.astype(jnp.int32)
    hi = hypothesis_indices.astype(jnp.int32)

    pooled = _pool_kernel(pi, hi, table_prem, table_hypo)

    w_scaled = W * (1.0 / _L)
    b2 = b.reshape(1, 3)
    bm = 2048
    probs = pl.pallas_call(
        _linear_body,
        grid=(_B // bm,),
        in_specs=[
            pl.BlockSpec((bm, 2 * _EMB), lambda i: (i, 0)),
            pl.BlockSpec((2 * _EMB, 3), lambda i: (0, 0)),
            pl.BlockSpec((1, 3), lambda i: (0, 0)),
        ],
        out_specs=pl.BlockSpec((bm, 3), lambda i: (i, 0)),
        out_shape=jax.ShapeDtypeStruct((_B, 3), jnp.float32),
    )(pooled, w_scaled, b2)
    return probs
